# Initial kernel scaffold; baseline (speedup 1.0000x reference)
import jax
import jax.numpy as jnp
from jax.experimental import pallas as pl


def kernel(Q, slab):
    raise NotImplementedError("write your pallas kernel here")


# speedup vs baseline: 2.2108x; 2.2108x over previous
import functools
import math

import jax
import jax.numpy as jnp
from jax.experimental import pallas as pl
from jax.experimental.pallas import tpu as pltpu

_N_Q = 6
_N_XI = 4
_NS = 8
_H = 32
_SR = _NS * _H

_ROW_WK, _ROW_PHASE, _ROW_B3 = 0, 8, 16
_ROW_W1, _ROW_W2 = 24, 56
_ROW_B1, _ROW_B2, _ROW_W3 = 88, 120, 152

_TILE = 2048

_FLOPS_PER_CFG = 2 * (_SR * _N_Q + _SR * 2 * _N_XI + _SR * _SR + _NS * _SR)
_TRANS_PER_CFG = 2 * _N_XI + 2 * _SR


def _fwd_kernel(qT_ref, wk2_ref, offs_ref, m1_ref, w2_ref, w3_ref,
                b1_ref, b2_ref, b3_ref, u_ref):
    z8 = jnp.dot(wk2_ref[...], qT_ref[...],
                 preferred_element_type=jnp.float32)
    sc = jnp.sin(z8 + offs_ref[...])
    h1 = jnp.tanh(jnp.dot(m1_ref[...], sc.astype(jnp.bfloat16),
                          preferred_element_type=jnp.float32) + b1_ref[...])
    h2 = jnp.tanh(jnp.dot(w2_ref[...], h1.astype(jnp.bfloat16),
                          preferred_element_type=jnp.float32) + b2_ref[...])
    u_ref[...] = jnp.dot(w3_ref[...], h2.astype(jnp.bfloat16),
                         preferred_element_type=jnp.float32) + b3_ref[0, 0]


@jax.jit
def kernel(Q, slab):
    wkT = slab[_ROW_WK:_ROW_WK + _N_XI, 0:_N_Q]
    phaseT = slab[_ROW_PHASE:_ROW_PHASE + _N_XI, 0:_NS]
    b3 = slab[_ROW_B3:_ROW_B3 + 1, 0:1]
    w1T = slab[_ROW_W1:_ROW_W1 + _H, 0:_N_XI]
    w2T = slab[_ROW_W2:_ROW_W2 + _H, 0:_H]
    b1T = slab[_ROW_B1:_ROW_B1 + _H, 0:1]
    b2T = slab[_ROW_B2:_ROW_B2 + _H, 0:1]
    w3c = slab[_ROW_W3:_ROW_W3 + _H, 0:1]

    wk2 = jnp.concatenate([wkT, wkT], axis=0)
    offs = jnp.concatenate([jnp.zeros((_N_XI, 1), jnp.float32),
                            jnp.full((_N_XI, 1), 0.5 * math.pi, jnp.float32)],
                           axis=0)

    cosP = jnp.cos(phaseT).T[:, None, :]
    sinP = jnp.sin(phaseT).T[:, None, :]
    m1 = jnp.concatenate([w1T[None, :, :] * cosP,
                          w1T[None, :, :] * sinP], axis=-1)
    m1 = m1.reshape(_SR, 2 * _N_XI).astype(jnp.bfloat16)

    eye = jnp.eye(_NS, dtype=jnp.float32)
    w2bd = jnp.kron(eye, w2T).astype(jnp.bfloat16)
    w3bd = jnp.kron(eye, w3c.T).astype(jnp.bfloat16)
    b1r = jnp.tile(b1T, (_NS, 1))
    b2r = jnp.tile(b2T, (_NS, 1))

    B = Q.shape[0]
    b_pad = ((B + _TILE - 1) // _TILE) * _TILE
    grid = b_pad // _TILE
    Qp = jnp.zeros((b_pad, _N_Q), jnp.float32).at[:B].set(Q.astype(jnp.float32))
    qT = Qp.T

    whole = lambda shp: pl.BlockSpec(shp, lambda i: (0, 0))
    out = pl.pallas_call(
        _fwd_kernel,
        out_shape=jax.ShapeDtypeStruct((_NS, b_pad), jnp.float32),
        grid=(grid,),
        in_specs=[
            pl.BlockSpec((_N_Q, _TILE), lambda i: (0, i)),
            whole((_NS, _N_Q)), whole((_NS, 1)),
            whole((_SR, 2 * _N_XI)), whole((_SR, _SR)), whole((_NS, _SR)),
            whole((_SR, 1)), whole((_SR, 1)), whole((1, 1)),
        ],
        out_specs=pl.BlockSpec((_NS, _TILE), lambda i: (0, i)),
        compiler_params=pltpu.CompilerParams(
            dimension_semantics=("parallel",)),
        cost_estimate=pl.CostEstimate(
            flops=_FLOPS_PER_CFG * b_pad,
            transcendentals=_TRANS_PER_CFG * b_pad,
            bytes_accessed=4 * (_N_Q + _NS) * b_pad + 2 * _SR * _SR),
    )(qT, wk2, offs, m1, w2bd, w3bd, b1r, b2r, b3)
    return out[:, :B].T


# speedup vs baseline: 2.6120x; 1.1815x over previous
import functools
import math

import jax
import jax.numpy as jnp
from jax.experimental import pallas as pl
from jax.experimental.pallas import tpu as pltpu

_N_Q = 6
_N_XI = 4
_NS = 8
_H = 32
_SR = _NS * _H

_ROW_WK, _ROW_PHASE, _ROW_B3 = 0, 8, 16
_ROW_W1, _ROW_W2 = 24, 56
_ROW_B1, _ROW_B2, _ROW_W3 = 88, 120, 152

_TILE = 4096
_CHUNK = 2048

_FLOPS_PER_CFG = 2 * (_SR * _N_Q + _SR * 2 * _N_XI + _SR * _SR + _NS * _SR)
_TRANS_PER_CFG = 2 * _N_XI + 2 * _SR

_INV_PI = 0.31830988618379067
_PI_HI = 3.140625
_PI_LO = 9.676535897932e-4
_C9 = 2.7557319e-6
_C7 = -1.9841270e-4
_C5 = 8.3333333e-3
_C3 = -0.16666667


def _fast_sin(x):
    kf = jnp.floor(x * _INV_PI + 0.5)
    r = x - kf * _PI_HI
    r = r - kf * _PI_LO
    r2 = r * r
    p = _C9 * r2 + _C7
    p = p * r2 + _C5
    p = p * r2 + _C3
    s = r * (p * r2 + 1.0)
    sbit = jax.lax.shift_left(jax.lax.bitwise_and(kf.astype(jnp.int32), 1), 31)
    bits = jax.lax.bitwise_xor(jax.lax.bitcast_convert_type(s, jnp.int32), sbit)
    return jax.lax.bitcast_convert_type(bits, jnp.float32)


def _fwd_kernel(qT_ref, wk2_ref, offs_ref, m1_ref, w2_ref, w3_ref,
                b1_ref, b2_ref, b3_ref, u_ref):
    for c in range(_TILE // _CHUNK):
        lo = c * _CHUNK
        z8 = jnp.dot(wk2_ref[...], qT_ref[:, lo:lo + _CHUNK],
                     preferred_element_type=jnp.float32)
        sc = _fast_sin(z8 + offs_ref[...])
        h1 = jnp.tanh(jnp.dot(m1_ref[...], sc.astype(jnp.bfloat16),
                              preferred_element_type=jnp.float32) + b1_ref[...])
        h2 = jnp.tanh(jnp.dot(w2_ref[...], h1.astype(jnp.bfloat16),
                              preferred_element_type=jnp.float32) + b2_ref[...])
        u_ref[:, lo:lo + _CHUNK] = jnp.dot(
            w3_ref[...], h2.astype(jnp.bfloat16),
            preferred_element_type=jnp.float32) + b3_ref[0, 0]


@jax.jit
def kernel(Q, slab):
    wkT = slab[_ROW_WK:_ROW_WK + _N_XI, 0:_N_Q]
    phaseT = slab[_ROW_PHASE:_ROW_PHASE + _N_XI, 0:_NS]
    b3 = slab[_ROW_B3:_ROW_B3 + 1, 0:1]
    w1T = slab[_ROW_W1:_ROW_W1 + _H, 0:_N_XI]
    w2T = slab[_ROW_W2:_ROW_W2 + _H, 0:_H]
    b1T = slab[_ROW_B1:_ROW_B1 + _H, 0:1]
    b2T = slab[_ROW_B2:_ROW_B2 + _H, 0:1]
    w3c = slab[_ROW_W3:_ROW_W3 + _H, 0:1]

    wk2 = jnp.concatenate([wkT, wkT], axis=0)
    offs = jnp.concatenate([jnp.zeros((_N_XI, 1), jnp.float32),
                            jnp.full((_N_XI, 1), 0.5 * math.pi, jnp.float32)],
                           axis=0)

    cosP = jnp.cos(phaseT).T[:, None, :]
    sinP = jnp.sin(phaseT).T[:, None, :]
    m1 = jnp.concatenate([w1T[None, :, :] * cosP,
                          w1T[None, :, :] * sinP], axis=-1)
    m1 = m1.reshape(_SR, 2 * _N_XI).astype(jnp.bfloat16)

    eye = jnp.eye(_NS, dtype=jnp.float32)
    w2bd = jnp.kron(eye, w2T).astype(jnp.bfloat16)
    w3bd = jnp.kron(eye, w3c.T).astype(jnp.bfloat16)
    b1r = jnp.tile(b1T, (_NS, 1))
    b2r = jnp.tile(b2T, (_NS, 1))

    B = Q.shape[0]
    b_pad = ((B + _TILE - 1) // _TILE) * _TILE
    grid = b_pad // _TILE
    if b_pad == B:
        Qp = Q.astype(jnp.float32)
    else:
        Qp = jnp.zeros((b_pad, _N_Q), jnp.float32).at[:B].set(
            Q.astype(jnp.float32))
    qT = Qp.T

    whole = lambda shp: pl.BlockSpec(shp, lambda i: (0, 0))
    out = pl.pallas_call(
        _fwd_kernel,
        out_shape=jax.ShapeDtypeStruct((_NS, b_pad), jnp.float32),
        grid=(grid,),
        in_specs=[
            pl.BlockSpec((_N_Q, _TILE), lambda i: (0, i)),
            whole((_NS, _N_Q)), whole((_NS, 1)),
            whole((_SR, 2 * _N_XI)), whole((_SR, _SR)), whole((_NS, _SR)),
            whole((_SR, 1)), whole((_SR, 1)), whole((1, 1)),
        ],
        out_specs=pl.BlockSpec((_NS, _TILE), lambda i: (0, i)),
        compiler_params=pltpu.CompilerParams(
            dimension_semantics=("parallel",)),
        cost_estimate=pl.CostEstimate(
            flops=_FLOPS_PER_CFG * b_pad,
            transcendentals=_TRANS_PER_CFG * b_pad,
            bytes_accessed=4 * (_N_Q + _NS) * b_pad + 2 * _SR * _SR),
    )(qT, wk2, offs, m1, w2bd, w3bd, b1r, b2r, b3)
    return out[:, :B].T
